# baseline (speedup 1.0000x reference)
import jax
import jax.numpy as jnp
from jax import lax


@jax.jit
def _run(inp, idx, src):
  M, D = inp.shape
  B = idx.shape[0]
  col = jnp.broadcast_to(jnp.arange(D, dtype=jnp.int32), (B, D))
  key = (idx.astype(jnp.int32) * D + col).reshape(-1)
  pos = jnp.arange(B * D, dtype=jnp.int32)
  k_s, p_s = lax.sort((key, pos), num_keys=1, is_stable=False)
  nxt = jnp.concatenate([k_s[1:], jnp.full((1,), -1, k_s.dtype)])
  is_last = k_s != nxt
  tgt = jnp.where(is_last, k_s, M * D)
  out_pad = jnp.concatenate([inp.reshape(-1), jnp.zeros((1,), jnp.float32)])
  out_pad = out_pad.at[tgt].set(src.reshape(-1)[p_s])
  return out_pad[: M * D].reshape(M, D)


def kernel(input, dim, index, src):
  del dim
  return _run(input, index, src)


# speedup vs baseline: 1.9834x; 1.9834x over previous
import functools

import jax
import jax.numpy as jnp
from jax import lax
from jax.experimental import pallas as pl
from jax.experimental.pallas import tpu as pltpu
from jax.experimental.pallas import tpu_sc as plsc

_PADW = 16384


def _build_scatter(M, D, B):
  NC, NS = 2, 16
  NW = NC * NS
  E = B * D
  PW = E // NW
  CHK = 8192
  NCH = PW // CHK

  mesh = plsc.VectorSubcoreMesh(core_axis_name="c", subcore_axis_name="s")

  def body(refs):
    w_ref, tgt_ref, val_ref = refs

    @pl.core_map(
        mesh,
        scratch_shapes=[
            pltpu.VMEM((CHK,), jnp.int32),
            pltpu.VMEM((CHK,), jnp.int32),
            pltpu.VMEM((CHK,), jnp.float32),
            pltpu.VMEM((CHK,), jnp.float32),
            pltpu.SemaphoreType.DMA,
        ],
    )
    def _(tgt_v0, tgt_v1, val_v0, val_v1, sem):
      core = lax.axis_index("c")
      sub = lax.axis_index("s")
      w = sub * NC + core
      base = w * PW
      bufs = [(tgt_v0, val_v0), (tgt_v1, val_v1)]
      descs = [None, None]
      for t in range(NCH):
        b = t % 2
        tgt_v, val_v = bufs[b]
        if descs[b] is not None:
          descs[b].wait()
        o = base + t * CHK
        pltpu.sync_copy(tgt_ref.at[pl.ds(o, CHK)], tgt_v)
        pltpu.sync_copy(val_ref.at[pl.ds(o, CHK)], val_v)
        descs[b] = pltpu.async_copy(val_v, w_ref.at[tgt_v], sem)
      for d in descs:
        if d is not None:
          d.wait()

  return pl.run_state(body)


@jax.jit
def _run(inp, idx, src):
  M, D = inp.shape
  B = idx.shape[0]
  E = B * D
  col = jnp.broadcast_to(jnp.arange(D, dtype=idx.dtype), (B, D))
  key = (idx * D + col).reshape(-1)
  k_s, v_s = lax.sort((key, src.reshape(-1)), num_keys=1, is_stable=False)
  nxt = jnp.concatenate([k_s[1:], jnp.full((1,), -1, k_s.dtype)])
  spread = M * D + (jnp.arange(E, dtype=jnp.int32) & (_PADW - 1))
  tgt = jnp.where(k_s != nxt, k_s, spread)
  w0 = jnp.concatenate([inp.reshape(-1), jnp.zeros((_PADW,), jnp.float32)])
  w, _, _ = _build_scatter(M, D, B)((w0, tgt, v_s))
  return w[: M * D].reshape(M, D)


def kernel(input, dim, index, src):
  del dim
  return _run(input, index, src)


# speedup vs baseline: 5.2753x; 2.6598x over previous
import jax
import jax.numpy as jnp
from jax import lax
from jax.experimental import pallas as pl
from jax.experimental.pallas import tpu as pltpu
from jax.experimental.pallas import tpu_sc as plsc

_RB = 80
_W = 8192
_SENT = 2**31 - 1


def _build(M, D, B):
  NC, NS, L = 2, 16, 16
  NW = NC * NS
  BS = _RB * D
  NBLK = (M // _RB)
  SH = (D - 1).bit_length()

  mesh = plsc.VectorSubcoreMesh(core_axis_name="c", subcore_axis_name="s")

  @pl.kernel(
      out_type=jax.ShapeDtypeStruct((M, D), jnp.float32),
      mesh=mesh,
      compiler_params=pltpu.CompilerParams(needs_layout_passes=False),
      scratch_types=[
          pltpu.VMEM((_RB, D), jnp.float32),
          pltpu.VMEM((_W,), jnp.int32),
          pltpu.VMEM((_W,), jnp.float32),
          pltpu.VMEM((NBLK + 6,), jnp.int32),
      ],
  )
  def k(inp, tgt, val, rp, out, blk_v, cell_v, valw_v, rp_v):
    core = lax.axis_index("c")
    sub = lax.axis_index("s")
    wid = sub * NC + core
    lanes = lax.iota(jnp.int32, L)

    pltpu.sync_copy(rp, rp_v)

    @pl.loop(wid, NBLK, step=NW)
    def _blk(b):
      r0 = b * _RB
      gbase = b * BS
      pltpu.sync_copy(inp.at[pl.ds(r0, _RB), :], blk_v)

      b0 = pl.multiple_of((b >> 3) << 3, 8)
      q = b - b0
      rvec = rp_v[pl.ds(b0, L)]
      neg = jnp.full((L,), -(2**31), jnp.int32)
      p0 = jnp.max(jnp.where(lanes == q, rvec, neg))
      p1 = jnp.max(jnp.where(lanes == q + 1, rvec, neg))
      o0 = pl.multiple_of((p0 >> 3) << 3, 8)
      nw = (p1 - o0 + _W - 1) // _W

      @pl.loop(0, nw)
      def _win(w2):
        pltpu.sync_copy(tgt.at[pl.ds(o0 + w2 * _W, _W)], cell_v)
        pltpu.sync_copy(val.at[pl.ds(o0 + w2 * _W, _W)], valw_v)

        @pl.loop(0, _W // L, unroll=8)
        def _vec(v):
          s = pl.ds(v * L, L)
          cells = cell_v[s]
          vals = valw_v[s]
          loc = cells - gbase
          mask = (loc >= 0) & (loc < BS)
          locc = jnp.where(mask, loc, 0)
          plsc.store_scatter(
              blk_v, [locc >> SH, locc & (D - 1)], vals, mask=mask)

      pltpu.sync_copy(blk_v, out.at[pl.ds(r0, _RB), :])

  return k


@jax.jit
def _run(inp, idx, src):
  M, D = inp.shape
  B = idx.shape[0]
  E = B * D
  BS = _RB * D
  NBLK = M // _RB
  col = jnp.broadcast_to(jnp.arange(D, dtype=idx.dtype), (B, D))
  key = (idx * D + col).reshape(-1)
  k_s, v_s = lax.sort((key, src.reshape(-1)), num_keys=1, is_stable=False)
  nxt = jnp.concatenate([k_s[1:], jnp.full((1,), -1, k_s.dtype)])
  tgt = jnp.where(k_s != nxt, k_s, _SENT)
  bounds = jnp.arange(NBLK + 1, dtype=jnp.int32) * BS
  rp = jnp.searchsorted(k_s, bounds, side="left").astype(jnp.int32)
  rp = jnp.concatenate([rp, jnp.full((5,), E, jnp.int32)])
  tgt = jnp.concatenate([tgt, jnp.full((_W + 8,), _SENT, jnp.int32)])
  val = jnp.concatenate([v_s, jnp.zeros((_W + 8,), jnp.float32)])
  return _build(M, D, B)(inp, tgt, val, rp)


def kernel(input, dim, index, src):
  del dim
  return _run(input, index, src)


# speedup vs baseline: 5.3011x; 1.0049x over previous
import jax
import jax.numpy as jnp
from jax import lax
from jax.experimental import pallas as pl
from jax.experimental.pallas import tpu as pltpu
from jax.experimental.pallas import tpu_sc as plsc

_RB = 80
_W = 8192
_SENT = 2**31 - 1


def _build(M, D, B):
  NC, NS, L = 2, 16, 16
  NW = NC * NS
  BS = _RB * D
  NBLK = (M // _RB)
  SH = (D - 1).bit_length()
  E = B * D

  mesh = plsc.VectorSubcoreMesh(core_axis_name="c", subcore_axis_name="s")

  @pl.kernel(
      out_type=jax.ShapeDtypeStruct((M, D), jnp.float32),
      mesh=mesh,
      compiler_params=pltpu.CompilerParams(needs_layout_passes=False),
      scratch_types=[
          pltpu.VMEM((_RB, D), jnp.float32),
          pltpu.VMEM((_W,), jnp.int32),
          pltpu.VMEM((_W,), jnp.float32),
          pltpu.VMEM((NBLK + 6,), jnp.int32),
      ],
  )
  def k(inp, tgt, val, rp, out, blk_v, cell_v, valw_v, rp_v):
    core = lax.axis_index("c")
    sub = lax.axis_index("s")
    wid = sub * NC + core
    lanes = lax.iota(jnp.int32, L)

    pltpu.sync_copy(rp, rp_v)

    @pl.loop(wid, NBLK, step=NW)
    def _blk(b):
      r0 = b * _RB
      gbase = b * BS
      pltpu.sync_copy(inp.at[pl.ds(r0, _RB), :], blk_v)

      b0 = pl.multiple_of((b >> 3) << 3, 8)
      q = b - b0
      rvec = rp_v[pl.ds(b0, L)]
      neg = jnp.full((L,), -(2**31), jnp.int32)
      p0 = jnp.max(jnp.where(lanes == q, rvec, neg))
      p1 = jnp.max(jnp.where(lanes == q + 1, rvec, neg))
      o0 = pl.multiple_of((p0 >> 3) << 3, 8)
      nw = (p1 - o0 + _W - 1) // _W

      @pl.loop(0, nw)
      def _win(w2):
        sw = pl.multiple_of(jnp.minimum(o0 + w2 * _W, E - _W), 8)
        pltpu.sync_copy(tgt.at[pl.ds(sw, _W)], cell_v)
        pltpu.sync_copy(val.at[pl.ds(sw, _W)], valw_v)

        @pl.loop(0, _W // L, unroll=8)
        def _vec(v):
          s = pl.ds(v * L, L)
          cells = cell_v[s]
          vals = valw_v[s]
          loc = cells - gbase
          mask = (loc >= 0) & (loc < BS)
          locc = jnp.where(mask, loc, 0)
          plsc.store_scatter(
              blk_v, [locc >> SH, locc & (D - 1)], vals, mask=mask)

      pltpu.sync_copy(blk_v, out.at[pl.ds(r0, _RB), :])

  return k


@jax.jit
def _run(inp, idx, src):
  M, D = inp.shape
  B = idx.shape[0]
  E = B * D
  BS = _RB * D
  NBLK = M // _RB
  col = jnp.broadcast_to(jnp.arange(D, dtype=idx.dtype), (B, D))
  key = (idx * D + col).reshape(-1)
  k_s, v_s = lax.sort((key, src.reshape(-1)), num_keys=1, is_stable=False)
  nxt = jnp.concatenate([k_s[1:], jnp.full((1,), -1, k_s.dtype)])
  tgt = jnp.where(k_s != nxt, k_s, _SENT)
  bounds = jnp.arange(NBLK + 1, dtype=jnp.int32) * BS
  rp = jnp.searchsorted(k_s, bounds, side="left").astype(jnp.int32)
  rp = jnp.concatenate([rp, jnp.full((5,), E, jnp.int32)])
  return _build(M, D, B)(inp, tgt, v_s, rp)


def kernel(input, dim, index, src):
  del dim
  return _run(input, index, src)


# speedup vs baseline: 5.3029x; 1.0003x over previous
import jax
import jax.numpy as jnp
from jax import lax
from jax.experimental import pallas as pl
from jax.experimental.pallas import tpu as pltpu
from jax.experimental.pallas import tpu_sc as plsc

_RB = 80
_W = 8192
_SENT = 2**31 - 1


def _build(M, D, B):
  NC, NS, L = 2, 16, 16
  NW = NC * NS
  BS = _RB * D
  NBLK = (M // _RB)
  SH = (D - 1).bit_length()
  E = B * D
  RPN = (NBLK + 17 + 7) // 8 * 8

  mesh = plsc.VectorSubcoreMesh(core_axis_name="c", subcore_axis_name="s")

  @pl.kernel(
      out_type=jax.ShapeDtypeStruct((M, D), jnp.float32),
      mesh=mesh,
      compiler_params=pltpu.CompilerParams(needs_layout_passes=False),
      scratch_types=[
          pltpu.VMEM((_RB, D), jnp.float32),
          pltpu.VMEM((_W,), jnp.int32),
          pltpu.VMEM((_W,), jnp.float32),
          pltpu.VMEM((RPN,), jnp.int32),
      ],
  )
  def k(inp, tgt, val, rp, out, blk_v, cell_v, valw_v, rp_v):
    core = lax.axis_index("c")
    sub = lax.axis_index("s")
    wid = sub * NC + core
    lanes = lax.iota(jnp.int32, L)

    pltpu.sync_copy(rp, rp_v)

    @pl.loop(wid, NBLK, step=NW)
    def _blk(b):
      r0 = b * _RB
      gbase = b * BS
      pltpu.sync_copy(inp.at[pl.ds(r0, _RB), :], blk_v)

      b0 = pl.multiple_of((b >> 3) << 3, 8)
      q = b - b0
      rvec = rp_v[pl.ds(b0, L)]
      neg = jnp.full((L,), -(2**31), jnp.int32)
      p0 = jnp.max(jnp.where(lanes == q, rvec, neg))
      p1 = jnp.max(jnp.where(lanes == q + 1, rvec, neg))
      o0 = pl.multiple_of((p0 >> 3) << 3, 8)
      nw = (p1 - o0 + _W - 1) // _W

      @pl.loop(0, nw)
      def _win(w2):
        sw = pl.multiple_of(jnp.minimum(o0 + w2 * _W, E - _W), 8)
        pltpu.sync_copy(tgt.at[pl.ds(sw, _W)], cell_v)
        pltpu.sync_copy(val.at[pl.ds(sw, _W)], valw_v)

        @pl.loop(0, _W // L, unroll=8)
        def _vec(v):
          s = pl.ds(v * L, L)
          cells = cell_v[s]
          vals = valw_v[s]
          loc = cells - gbase
          mask = (loc >= 0) & (loc < BS)
          locc = jnp.where(mask, loc, 0)
          plsc.store_scatter(
              blk_v, [locc >> SH, locc & (D - 1)], vals, mask=mask)

      pltpu.sync_copy(blk_v, out.at[pl.ds(r0, _RB), :])

  return k


@jax.jit
def _run(inp, idx, src):
  M, D = inp.shape
  B = idx.shape[0]
  E = B * D
  BS = _RB * D
  NBLK = M // _RB
  col = jnp.broadcast_to(jnp.arange(D, dtype=idx.dtype), (B, D))
  key = (idx * D + col).reshape(-1)
  k_s, v_s = lax.sort((key, src.reshape(-1)), num_keys=1, is_stable=False)
  nxt = jnp.concatenate([k_s[1:], jnp.full((1,), -1, k_s.dtype)])
  tgt = jnp.where(k_s != nxt, k_s, _SENT)
  bounds = jnp.arange(NBLK + 1, dtype=jnp.int32) * BS
  rp = jnp.searchsorted(k_s, bounds, side="left").astype(jnp.int32)
  RPN = (NBLK + 17 + 7) // 8 * 8
  rp = jnp.concatenate([rp, jnp.full((RPN - NBLK - 1,), E, jnp.int32)])
  return _build(M, D, B)(inp, tgt, v_s, rp)


def kernel(input, dim, index, src):
  del dim
  return _run(input, index, src)
